# Initial kernel scaffold; baseline (speedup 1.0000x reference)
import jax
import jax.numpy as jnp
from jax.experimental import pallas as pl


def kernel(x, X, Y, W):
    raise NotImplementedError("write your pallas kernel here")


# speedup vs baseline: 2.0156x; 2.0156x over previous
import functools
import math

import jax
import jax.numpy as jnp
from jax import lax
from jax.experimental import pallas as pl
from jax.experimental.pallas import tpu as pltpu
from jax.experimental.pallas import tpu_sc as plsc

_K = 8
_NBLK = 2048
_LANES = 16
_NTILES = 32
_PAD_X = 1e9
_NEG = -3.0e38


def _tc_body(nblk, k, x_ref, w_ref, xb_ref, yb_ref, vals_ref, labs_ref,
             logits2_ref):
    b = x_ref.shape[0]

    @pl.when(pl.program_id(0) == 0)
    def _():
        logits2_ref[...] = 2.0 * jnp.dot(
            x_ref[...].astype(jnp.bfloat16), w_ref[...].astype(jnp.bfloat16),
            preferred_element_type=jnp.float32)

    xb = xb_ref[...]
    ones = jnp.ones((1, xb.shape[1]), jnp.float32)
    ksq = lax.dot_general(ones, xb * xb, (((1,), (1,)), ((), ())),
                          preferred_element_type=jnp.float32,
                          precision=lax.Precision.HIGHEST)
    scores = lax.dot_general(logits2_ref[...].astype(jnp.bfloat16),
                             xb.astype(jnp.bfloat16), (((1,), (1,)), ((), ())),
                             preferred_element_type=jnp.float32) - ksq
    lane = lax.broadcasted_iota(jnp.int32, (b, nblk), 1)
    yb = yb_ref[...]
    d = scores
    vcols, lcols = [], []
    for _ in range(k):
        m = jnp.max(d, axis=1, keepdims=True)
        amin = jnp.min(jnp.where(d == m, lane, nblk), axis=1,
                       keepdims=True)
        sel = lane == amin
        lab = jnp.sum(jnp.where(sel, yb, 0.0), axis=1, keepdims=True)
        vcols.append(m)
        lcols.append(lab)
        d = jnp.where(sel, _NEG, d)
    vals_ref[0] = jnp.concatenate(vcols, axis=1)
    labs_ref[0] = jnp.concatenate(lcols, axis=1)


def _phase1(x, xdb, y, w):
    b, d = x.shape
    n = xdb.shape[0]
    nb = math.ceil(n / _NBLK)
    npad = nb * _NBLK
    xp = jnp.concatenate(
        [xdb, jnp.full((npad - n, d), _PAD_X, jnp.float32)], axis=0)
    yf = jnp.concatenate(
        [y.astype(jnp.float32), jnp.zeros((npad - n,), jnp.float32)]
    ).reshape(1, npad)
    vals, labs = pl.pallas_call(
        functools.partial(_tc_body, _NBLK, _K),
        grid=(nb,),
        in_specs=[
            pl.BlockSpec((b, d), lambda j: (0, 0)),
            pl.BlockSpec((d, d), lambda j: (0, 0)),
            pl.BlockSpec((_NBLK, d), lambda j: (j, 0)),
            pl.BlockSpec((1, _NBLK), lambda j: (0, j)),
        ],
        out_specs=[
            pl.BlockSpec((1, b, _K), lambda j: (j, 0, 0)),
            pl.BlockSpec((1, b, _K), lambda j: (j, 0, 0)),
        ],
        out_shape=[
            jax.ShapeDtypeStruct((nb, b, _K), jnp.float32),
            jax.ShapeDtypeStruct((nb, b, _K), jnp.float32),
        ],
        scratch_shapes=[pltpu.VMEM((b, d), jnp.float32)],
    )(x, w, xp, yf)
    vals = jnp.transpose(vals, (1, 0, 2)).reshape(b, nb * _K)
    labs = jnp.transpose(labs, (1, 0, 2)).reshape(b, nb * _K)
    return vals, labs


def _sc_body(qt, cand, vals_hbm, labs_hbm, out_hbm, vals_v, labs_v, votes_v):
    wid = lax.axis_index("s") * 2 + lax.axis_index("c")
    base = wid * qt
    pltpu.sync_copy(vals_hbm.at[pl.ds(base, qt), :], vals_v)
    pltpu.sync_copy(labs_hbm.at[pl.ds(base, qt), :], labs_v)
    lane = lax.iota(jnp.int32, _LANES)

    def q_body(g, i, votes_vec):
        q = g * _LANES + i
        r = jnp.full((_LANES,), _NEG, jnp.float32)
        rl = jnp.zeros((_LANES,), jnp.float32)
        for c in range(cand // _LANES):
            v = vals_v[q, pl.ds(c * _LANES, _LANES)]
            l = labs_v[q, pl.ds(c * _LANES, _LANES)]
            vs, ls = plsc.sort_key_val(v, l)
            keep = r >= vs
            nv = jnp.where(keep, r, vs)
            nl = jnp.where(keep, rl, ls)
            r, rl = plsc.sort_key_val(nv, nl, descending=True)
        vote = jnp.sum(jnp.where(lane < _K, rl, 0.0), axis=0) * (1.0 / _K)
        return jnp.where(lane == i, vote, votes_vec)

    for g in range(qt // _LANES):
        votes_vec = lax.fori_loop(
            0, _LANES, functools.partial(q_body, g),
            jnp.zeros((_LANES,), jnp.float32))
        votes_v[pl.ds(g * _LANES, _LANES)] = votes_vec
    pltpu.sync_copy(votes_v, out_hbm.at[pl.ds(base, qt)])


def _phase2_sc(vals, labs):
    b, c = vals.shape
    cand = math.ceil(c / _LANES) * _LANES
    vals = jnp.pad(vals, ((0, 0), (0, cand - c)), constant_values=_NEG)
    labs = jnp.pad(labs, ((0, 0), (0, cand - c)))
    qt = b // _NTILES
    mesh = plsc.VectorSubcoreMesh(core_axis_name="c", subcore_axis_name="s")
    return pl.kernel(
        functools.partial(_sc_body, qt, cand),
        out_type=jax.ShapeDtypeStruct((b,), jnp.float32),
        mesh=mesh,
        scratch_types=[
            pltpu.VMEM((qt, cand), jnp.float32),
            pltpu.VMEM((qt, cand), jnp.float32),
            pltpu.VMEM((qt,), jnp.float32),
        ],
        compiler_params=pltpu.CompilerParams(needs_layout_passes=False),
    )(vals, labs)


def kernel(x, X, Y, W):
    vals, labs = _phase1(x, X, Y, W)
    votes = _phase2_sc(vals, labs)
    return jnp.pad(votes[:, None], ((0, 0), (0, 1)))


# speedup vs baseline: 2.5017x; 1.2412x over previous
import functools
import math

import jax
import jax.numpy as jnp
from jax import lax
from jax.experimental import pallas as pl
from jax.experimental.pallas import tpu as pltpu
from jax.experimental.pallas import tpu_sc as plsc

_K = 8
_NBLK = 2048
_LANES = 16
_NTILES = 32
_PAD_X = 1e9
_NEG = -3.0e38


def _tc_body(nblk, k, x_ref, w_ref, xb_ref, vals_ref, idx_ref, logits2_ref):
    b = x_ref.shape[0]

    @pl.when(pl.program_id(0) == 0)
    def _():
        logits2_ref[...] = 2.0 * jnp.dot(
            x_ref[...].astype(jnp.bfloat16), w_ref[...].astype(jnp.bfloat16),
            preferred_element_type=jnp.float32)

    xb = xb_ref[...]
    ones = jnp.ones((1, xb.shape[1]), jnp.float32)
    ksq = lax.dot_general(ones, xb * xb, (((1,), (1,)), ((), ())),
                          preferred_element_type=jnp.float32,
                          precision=lax.Precision.HIGHEST)
    scores = lax.dot_general(logits2_ref[...].astype(jnp.bfloat16),
                             xb.astype(jnp.bfloat16), (((1,), (1,)), ((), ())),
                             preferred_element_type=jnp.float32) - ksq
    lane = lax.broadcasted_iota(jnp.int32, (b, nblk), 1)
    d = scores
    vcols, icols = [], []
    for _ in range(k):
        m = jnp.max(d, axis=1, keepdims=True)
        amin = jnp.min(jnp.where(d == m, lane, nblk), axis=1,
                       keepdims=True)
        vcols.append(m)
        icols.append(amin)
        d = jnp.where(lane == amin, _NEG, d)
    vals_ref[0] = jnp.concatenate(vcols, axis=1)
    idx_ref[0] = jnp.concatenate(icols, axis=1) + pl.program_id(0) * nblk


def _phase1(x, xdb, w):
    b, d = x.shape
    n = xdb.shape[0]
    nb = math.ceil(n / _NBLK)
    npad = nb * _NBLK
    xp = jnp.concatenate(
        [xdb, jnp.full((npad - n, d), _PAD_X, jnp.float32)], axis=0)
    vals, idx = pl.pallas_call(
        functools.partial(_tc_body, _NBLK, _K),
        grid=(nb,),
        in_specs=[
            pl.BlockSpec((b, d), lambda j: (0, 0)),
            pl.BlockSpec((d, d), lambda j: (0, 0)),
            pl.BlockSpec((_NBLK, d), lambda j: (j, 0)),
        ],
        out_specs=[
            pl.BlockSpec((1, b, _K), lambda j: (j, 0, 0)),
            pl.BlockSpec((1, b, _K), lambda j: (j, 0, 0)),
        ],
        out_shape=[
            jax.ShapeDtypeStruct((nb, b, _K), jnp.float32),
            jax.ShapeDtypeStruct((nb, b, _K), jnp.int32),
        ],
        scratch_shapes=[pltpu.VMEM((b, d), jnp.float32)],
    )(x, w, xp)
    vals = jnp.transpose(vals, (1, 0, 2)).reshape(b, nb * _K)
    idx = jnp.transpose(idx, (1, 0, 2)).reshape(b, nb * _K)
    return vals, idx


def _sc_body(qt, cand, vals_hbm, idx_hbm, y_hbm, out_hbm,
             vals_v, idx_v, top_idx_v, labs_v, votes_v, sem):
    wid = lax.axis_index("s") * 2 + lax.axis_index("c")
    base = wid * qt
    pltpu.sync_copy(vals_hbm.at[pl.ds(base, qt), :], vals_v)
    pltpu.sync_copy(idx_hbm.at[pl.ds(base, qt), :], idx_v)
    lane = lax.iota(jnp.int32, _LANES)

    def merge_body(q, carry):
        r = jnp.full((_LANES,), _NEG, jnp.float32)
        ri = jnp.zeros((_LANES,), jnp.int32)
        for c in range(cand // _LANES):
            v = vals_v[q, pl.ds(c * _LANES, _LANES)]
            ix = idx_v[q, pl.ds(c * _LANES, _LANES)]
            vs, ixs = plsc.sort_key_val(v, ix)
            keep = r >= vs
            nv = jnp.where(keep, r, vs)
            ni = jnp.where(keep, ri, ixs)
            r, ri = plsc.sort_key_val(nv, ni, descending=True)
        top_idx_v[pl.ds(q * _LANES, _LANES)] = ri
        return carry

    lax.fori_loop(0, qt, merge_body, 0)
    pltpu.async_copy(y_hbm.at[top_idx_v], labs_v, sem).wait()

    def vote_body(g, i, votes_vec):
        q = g * _LANES + i
        lv = labs_v[pl.ds(q * _LANES, _LANES)].astype(jnp.float32)
        vote = jnp.sum(jnp.where(lane < _K, lv, 0.0), axis=0) * (1.0 / _K)
        return jnp.where(lane == i, vote, votes_vec)

    for g in range(qt // _LANES):
        votes_vec = lax.fori_loop(
            0, _LANES, functools.partial(vote_body, g),
            jnp.zeros((_LANES,), jnp.float32))
        votes_v[pl.ds(g * _LANES, _LANES)] = votes_vec
    pltpu.sync_copy(votes_v, out_hbm.at[pl.ds(base, qt)])


def _phase2_sc(vals, idx, y):
    b, c = vals.shape
    cand = math.ceil(c / _LANES) * _LANES
    vals = jnp.pad(vals, ((0, 0), (0, cand - c)), constant_values=_NEG)
    idx = jnp.pad(idx, ((0, 0), (0, cand - c)))
    qt = b // _NTILES
    mesh = plsc.VectorSubcoreMesh(core_axis_name="c", subcore_axis_name="s")
    return pl.kernel(
        functools.partial(_sc_body, qt, cand),
        out_type=jax.ShapeDtypeStruct((b,), jnp.float32),
        mesh=mesh,
        scratch_types=[
            pltpu.VMEM((qt, cand), jnp.float32),
            pltpu.VMEM((qt, cand), jnp.int32),
            pltpu.VMEM((qt * _LANES,), jnp.int32),
            pltpu.VMEM((qt * _LANES,), jnp.int32),
            pltpu.VMEM((qt,), jnp.float32),
            pltpu.SemaphoreType.DMA,
        ],
        compiler_params=pltpu.CompilerParams(needs_layout_passes=False),
    )(vals, idx, y)


def kernel(x, X, Y, W):
    vals, idx = _phase1(x, X, W)
    votes = _phase2_sc(vals, idx, Y)
    return jnp.pad(votes[:, None], ((0, 0), (0, 1)))


# speedup vs baseline: 2.8996x; 1.1590x over previous
import functools
import math

import jax
import jax.numpy as jnp
from jax import lax
from jax.experimental import pallas as pl
from jax.experimental.pallas import tpu as pltpu
from jax.experimental.pallas import tpu_sc as plsc

_K = 8
_NBLK = 2048
_LANES = 16
_NTILES = 32
_PAD_X = 1e9
_NEG = -3.0e38


def _block_scores(x_ref, w_ref, xb_ref, logits2_ref):
    @pl.when(pl.program_id(0) == 0)
    def _():
        logits2_ref[...] = 2.0 * jnp.dot(
            x_ref[...].astype(jnp.bfloat16), w_ref[...].astype(jnp.bfloat16),
            preferred_element_type=jnp.float32)

    xb = xb_ref[...]
    ones = jnp.ones((1, xb.shape[1]), jnp.float32)
    ksq = lax.dot_general(ones, xb * xb, (((1,), (1,)), ((), ())),
                          preferred_element_type=jnp.float32,
                          precision=lax.Precision.HIGHEST)
    return lax.dot_general(logits2_ref[...].astype(jnp.bfloat16),
                           xb.astype(jnp.bfloat16), (((1,), (1,)), ((), ())),
                           preferred_element_type=jnp.float32) - ksq


def _k1_body(nblk, k, x_ref, w_ref, xb_ref, vals_ref, cnts_ref, logits2_ref):
    scores = _block_scores(x_ref, w_ref, xb_ref, logits2_ref)
    half = nblk // 2
    a = scores[:, :half]
    bq = scores[:, half:]
    p = jnp.maximum(a, bq)
    q = jnp.minimum(a, bq)
    vcols, ccols = [], []
    for _ in range(k):
        m = jnp.max(p, axis=1, keepdims=True)
        eqm = p == m
        cnt = jnp.sum(jnp.where(eqm, 1.0, 0.0), axis=1, keepdims=True)
        vcols.append(m)
        ccols.append(cnt)
        p = jnp.where(eqm, q, p)
        q = jnp.where(eqm, _NEG, q)
    vals_ref[0] = jnp.concatenate(vcols, axis=1)
    cnts_ref[0] = jnp.concatenate(ccols, axis=1)


def _k2_body(nblk, nb, k, x_ref, w_ref, xb_ref, yb_ref, t_ref, out_ref,
             logits2_ref, acc_ref):
    scores = _block_scores(x_ref, w_ref, xb_ref, logits2_ref)
    t = t_ref[...]
    yb = yb_ref[...]

    @pl.when(pl.program_id(0) == 0)
    def _():
        acc_ref[...] = jnp.zeros_like(acc_ref)

    gt = scores > t
    eq = scores == t
    lab_gt = jnp.sum(jnp.where(gt, yb, 0.0), axis=1, keepdims=True)
    cnt_gt = jnp.sum(jnp.where(gt, 1.0, 0.0), axis=1, keepdims=True)
    lab_eq = jnp.sum(jnp.where(eq, yb, 0.0), axis=1, keepdims=True)
    cnt_eq = jnp.sum(jnp.where(eq, 1.0, 0.0), axis=1, keepdims=True)
    acc_ref[...] += jnp.concatenate([lab_gt, cnt_gt, lab_eq, cnt_eq], axis=1)

    @pl.when(pl.program_id(0) == nb - 1)
    def _():
        acc = acc_ref[...]
        lg, cg = acc[:, 0:1], acc[:, 1:2]
        le, ce = acc[:, 2:3], acc[:, 3:4]
        out_ref[...] = (lg + (k - cg) * le / jnp.maximum(ce, 1.0)) * (1.0 / k)


def _pad_db(xdb):
    n, d = xdb.shape
    nb = math.ceil(n / _NBLK)
    xp = jnp.concatenate(
        [xdb, jnp.full((nb * _NBLK - n, d), _PAD_X, jnp.float32)], axis=0)
    return xp, nb


def _phase1(x, xp, w, nb):
    b, d = x.shape
    vals, cnts = pl.pallas_call(
        functools.partial(_k1_body, _NBLK, _K),
        grid=(nb,),
        in_specs=[
            pl.BlockSpec((b, d), lambda j: (0, 0)),
            pl.BlockSpec((d, d), lambda j: (0, 0)),
            pl.BlockSpec((_NBLK, d), lambda j: (j, 0)),
        ],
        out_specs=[
            pl.BlockSpec((1, b, _K), lambda j: (j, 0, 0)),
            pl.BlockSpec((1, b, _K), lambda j: (j, 0, 0)),
        ],
        out_shape=[
            jax.ShapeDtypeStruct((nb, b, _K), jnp.float32),
            jax.ShapeDtypeStruct((nb, b, _K), jnp.float32),
        ],
        scratch_shapes=[pltpu.VMEM((b, d), jnp.float32)],
    )(x, w, xp)
    vals = jnp.transpose(vals, (1, 0, 2)).reshape(b, nb * _K)
    cnts = jnp.transpose(cnts, (1, 0, 2)).reshape(b, nb * _K)
    return vals, cnts


def _sc_t_body(qt, cand, vals_hbm, cnts_hbm, out_hbm, vals_v, cnts_v, t_v):
    wid = lax.axis_index("s") * 2 + lax.axis_index("c")
    base = wid * qt
    pltpu.sync_copy(vals_hbm.at[pl.ds(base, qt), :], vals_v)
    pltpu.sync_copy(cnts_hbm.at[pl.ds(base, qt), :], cnts_v)
    lane = lax.iota(jnp.int32, _LANES)

    def q_body(g, i, t_vec):
        q = g * _LANES + i
        r = jnp.full((_LANES,), _NEG, jnp.float32)
        rc = jnp.zeros((_LANES,), jnp.float32)
        for c in range(cand // _LANES):
            v = vals_v[q, pl.ds(c * _LANES, _LANES)]
            cn = cnts_v[q, pl.ds(c * _LANES, _LANES)]
            vs, cs_ = plsc.sort_key_val(v, cn)
            keep = r >= vs
            nv = jnp.where(keep, r, vs)
            nc = jnp.where(keep, rc, cs_)
            r, rc = plsc.sort_key_val(nv, nc, descending=True)
        csum = plsc.cumsum(rc)
        t = jnp.max(jnp.where(csum >= float(_K), r, _NEG), axis=0)
        return jnp.where(lane == i, t, t_vec)

    for g in range(qt // _LANES):
        t_vec = lax.fori_loop(
            0, _LANES, functools.partial(q_body, g),
            jnp.zeros((_LANES,), jnp.float32))
        t_v[pl.ds(g * _LANES, _LANES)] = t_vec
    pltpu.sync_copy(t_v, out_hbm.at[pl.ds(base, qt)])


def _phase2_sc(vals, cnts):
    b, c = vals.shape
    cand = math.ceil(c / _LANES) * _LANES
    vals = jnp.pad(vals, ((0, 0), (0, cand - c)), constant_values=_NEG)
    cnts = jnp.pad(cnts, ((0, 0), (0, cand - c)))
    qt = b // _NTILES
    mesh = plsc.VectorSubcoreMesh(core_axis_name="c", subcore_axis_name="s")
    return pl.kernel(
        functools.partial(_sc_t_body, qt, cand),
        out_type=jax.ShapeDtypeStruct((b,), jnp.float32),
        mesh=mesh,
        scratch_types=[
            pltpu.VMEM((qt, cand), jnp.float32),
            pltpu.VMEM((qt, cand), jnp.float32),
            pltpu.VMEM((qt,), jnp.float32),
        ],
        compiler_params=pltpu.CompilerParams(needs_layout_passes=False),
    )(vals, cnts)


def _phase3(x, xp, y, w, t, nb):
    b, d = x.shape
    npad = nb * _NBLK
    yf = jnp.concatenate(
        [y.astype(jnp.float32), jnp.zeros((npad - y.shape[0],), jnp.float32)]
    ).reshape(1, npad)
    return pl.pallas_call(
        functools.partial(_k2_body, _NBLK, nb, _K),
        grid=(nb,),
        in_specs=[
            pl.BlockSpec((b, d), lambda j: (0, 0)),
            pl.BlockSpec((d, d), lambda j: (0, 0)),
            pl.BlockSpec((_NBLK, d), lambda j: (j, 0)),
            pl.BlockSpec((1, _NBLK), lambda j: (0, j)),
            pl.BlockSpec((b, 1), lambda j: (0, 0)),
        ],
        out_specs=pl.BlockSpec((b, 1), lambda j: (0, 0)),
        out_shape=jax.ShapeDtypeStruct((b, 1), jnp.float32),
        scratch_shapes=[pltpu.VMEM((b, d), jnp.float32),
                        pltpu.VMEM((b, 4), jnp.float32)],
    )(x, w, xp, yf, t)


def kernel(x, X, Y, W):
    xp, nb = _pad_db(X)
    vals, cnts = _phase1(x, xp, W, nb)
    t = _phase2_sc(vals, cnts)
    votes = _phase3(x, xp, Y, W, t[:, None], nb)
    return jnp.pad(votes, ((0, 0), (0, 1)))
